# baseline (speedup 1.0000x reference)
import functools

import jax
import jax.numpy as jnp
from jax import lax
from jax.experimental import pallas as pl
from jax.experimental.pallas import tpu as pltpu
from jax.experimental.pallas import tpu_sc as plsc

_NC = 2
_NS = 16
_NW = _NC * _NS
_NBUF = 10
_SPLIT = 2


def _sc_gather(xt, table):
    l, b = xt.shape
    d = table.shape[1]
    bw = b // _NW
    cw = bw // _SPLIT
    n_chunks = l * _SPLIT
    n_groups = n_chunks // _NBUF
    mesh = plsc.VectorSubcoreMesh(core_axis_name="c", subcore_axis_name="s")

    @functools.partial(
        pl.kernel,
        out_type=jax.ShapeDtypeStruct((l, b, d), jnp.float32),
        mesh=mesh,
        scratch_types=[
            pltpu.VMEM((l, bw), jnp.int32),
            [pltpu.VMEM((cw, d), jnp.float32) for _ in range(_NBUF)],
            [pltpu.SemaphoreType.DMA for _ in range(_NBUF)],
            [pltpu.SemaphoreType.DMA for _ in range(_NBUF)],
            pltpu.SemaphoreType.DMA,
        ],
    )
    def k(xt_hbm, tab_hbm, out_hbm, idx_v, bufs, gsems, osems, isem):
        wid = lax.axis_index("s") * _NC + lax.axis_index("c")
        b0 = wid * bw
        head = 8
        pltpu.async_copy(
            xt_hbm.at[pl.ds(0, head), pl.ds(b0, bw)],
            idx_v.at[pl.ds(0, head)], isem,
        ).wait()

        def idx_ref(j):
            return idx_v.at[j // _SPLIT, pl.ds((j % _SPLIT) * cw, cw)]

        def gather_start(j, bi):
            pltpu.async_copy(tab_hbm.at[idx_ref(j)], bufs[bi], gsems[bi])

        def gather_wait(j, bi):
            pltpu.make_async_copy(tab_hbm.at[idx_ref(j)], bufs[bi], gsems[bi]).wait()

        def out_ref(j):
            return out_hbm.at[j // _SPLIT, pl.ds(b0 + (j % _SPLIT) * cw, cw)]

        for bi in range(_NBUF):
            gather_start(bi, bi)

        tail = pltpu.async_copy(
            xt_hbm.at[pl.ds(head, l - head), pl.ds(b0, bw)],
            idx_v.at[pl.ds(head, l - head)], isem,
        )
        tail.wait()

        @pl.loop(0, n_groups)
        def _(g):
            j0 = g * _NBUF
            for bi in range(_NBUF):
                gather_wait(j0 + bi, bi)
                pltpu.async_copy(bufs[bi], out_ref(j0 + bi), osems[bi])

            @pl.when(g < n_groups - 1)
            def _():
                for bi in range(_NBUF):
                    pltpu.make_async_copy(
                        bufs[bi], out_ref(j0 + bi), osems[bi]
                    ).wait()
                    gather_start(j0 + _NBUF + bi, bi)

        for bi in range(_NBUF):
            j = (n_groups - 1) * _NBUF + bi
            pltpu.make_async_copy(bufs[bi], out_ref(j), osems[bi]).wait()

    return k(xt, table)


def kernel(x, embedding_weight):
    out_lbd = _sc_gather(x.T, embedding_weight)
    return jnp.transpose(out_lbd, (1, 0, 2))


# speedup vs baseline: 1.0106x; 1.0106x over previous
import functools

import jax
import jax.numpy as jnp
from jax import lax
from jax.experimental import pallas as pl
from jax.experimental.pallas import tpu as pltpu
from jax.experimental.pallas import tpu_sc as plsc

_NC = 2
_NS = 16
_NW = _NC * _NS
_NBUF = 10
_SPLIT = 2


def _sc_gather(xt, table):
    l, b = xt.shape
    d = table.shape[1]
    bw = b // _NW
    cw = bw // _SPLIT
    n_chunks = l * _SPLIT
    n_groups = n_chunks // _NBUF
    mesh = plsc.VectorSubcoreMesh(core_axis_name="c", subcore_axis_name="s")

    @functools.partial(
        pl.kernel,
        out_type=jax.ShapeDtypeStruct((l, b, d), jnp.float32),
        mesh=mesh,
        scratch_types=[
            pltpu.VMEM((l, bw), jnp.int32),
            [pltpu.VMEM((cw, d), jnp.float32) for _ in range(_NBUF)],
            [pltpu.SemaphoreType.DMA for _ in range(_NBUF)],
            [pltpu.SemaphoreType.DMA for _ in range(_NBUF)],
        ],
    )
    def k(xt_hbm, tab_hbm, out_hbm, idx_v, bufs, gsems, osems):
        wid = lax.axis_index("s") * _NC + lax.axis_index("c")
        b0 = wid * bw
        pltpu.sync_copy(xt_hbm.at[:, pl.ds(b0, bw)], idx_v)

        def idx_ref(j):
            return idx_v.at[j // _SPLIT, pl.ds((j % _SPLIT) * cw, cw)]

        def gather_start(j, bi):
            pltpu.async_copy(tab_hbm.at[idx_ref(j)], bufs[bi], gsems[bi])

        def gather_wait(j, bi):
            pltpu.make_async_copy(tab_hbm.at[idx_ref(j)], bufs[bi], gsems[bi]).wait()

        def out_ref(j):
            return out_hbm.at[j // _SPLIT, pl.ds(b0 + (j % _SPLIT) * cw, cw)]

        for bi in range(_NBUF):
            gather_start(bi, bi)

        @pl.loop(0, n_groups)
        def _(g):
            j0 = g * _NBUF
            for bi in range(_NBUF):
                gather_wait(j0 + bi, bi)
                pltpu.async_copy(bufs[bi], out_ref(j0 + bi), osems[bi])

            @pl.when(g < n_groups - 1)
            def _():
                for bi in range(_NBUF):
                    pltpu.make_async_copy(
                        bufs[bi], out_ref(j0 + bi), osems[bi]
                    ).wait()
                    gather_start(j0 + _NBUF + bi, bi)

        for bi in range(_NBUF):
            j = (n_groups - 1) * _NBUF + bi
            pltpu.make_async_copy(bufs[bi], out_ref(j), osems[bi]).wait()

    return k(xt, table)


def kernel(x, embedding_weight):
    out_lbd = _sc_gather(x.T, embedding_weight)
    return jnp.transpose(out_lbd, (1, 0, 2))
